# baseline (speedup 1.0000x reference)
import functools
import math

import jax
import jax.numpy as jnp
from jax import lax
from jax.experimental import pallas as pl
from jax.experimental.pallas import tpu as pltpu
from jax.experimental.pallas import tpu_sc as plsc

_L = 16


def _sc_body(nimg, h, w, xh, zh, ph, xb0, xb1, zb0, zb1,
             s_in0, s_in1, s_out0, s_out1, accv):
    h2, w2 = h // 2, w // 2
    hh = h // 2
    hh2 = hh // 2
    wid = lax.axis_index("s") * 2 + lax.axis_index("c")
    base = wid * nimg
    lane = lax.iota(jnp.int32, _L)
    evens = [lane * 2 + j * (2 * _L) for j in range(w2 // _L)]

    def compute(xb, zb, accs):
        @plsc.parallel_loop(0, hh2, 1, unroll=2, carry=accs)
        def rp_body(rp, accs):
            aa, ab, ac, ad = accs
            r0 = jnp.full((_L,), 2 * rp, jnp.int32)
            r1 = r0 + 1
            for j in range(w2 // _L):
                odd = evens[j] + 1
                e0 = plsc.load_gather(xb, [r0, evens[j]])
                o0 = plsc.load_gather(xb, [r0, odd])
                e1 = plsc.load_gather(xb, [r1, evens[j]])
                o1 = plsc.load_gather(xb, [r1, odd])
                z = jnp.maximum(jnp.maximum(e0, o0), jnp.maximum(e1, o1))
                zb[rp, pl.ds(j * _L, _L)] = z
                d0 = z - e0
                d1 = z - o0
                d2 = z - e1
                d3 = z - o1
                aa = aa + d0 * d0
                ab = ab + d1 * d1
                ac = ac + d2 * d2
                ad = ad + d3 * d3
            return (aa, ab, ac, ad)
        return rp_body

    pltpu.make_async_copy(xh.at[base, pl.ds(0, hh)], xb0, s_in0).start()

    def step(g, acc):
        img = base + g
        pltpu.make_async_copy(xh.at[img, pl.ds(hh, hh)], xb1, s_in1).start()
        pltpu.make_async_copy(xh.at[img, pl.ds(0, hh)], xb0, s_in0).wait()

        @pl.when(g > 0)
        def _():
            pltpu.make_async_copy(zb0, zh.at[img - 1, pl.ds(0, hh2)], s_out0).wait()

        acc = compute(xb0, zb0, acc)
        pltpu.make_async_copy(zb0, zh.at[img, pl.ds(0, hh2)], s_out0).start()

        @pl.when(g + 1 < nimg)
        def _():
            pltpu.make_async_copy(xh.at[img + 1, pl.ds(0, hh)], xb0, s_in0).start()

        pltpu.make_async_copy(xh.at[img, pl.ds(hh, hh)], xb1, s_in1).wait()

        @pl.when(g > 0)
        def _():
            pltpu.make_async_copy(zb1, zh.at[img - 1, pl.ds(hh2, hh2)], s_out1).wait()

        acc = compute(xb1, zb1, acc)
        pltpu.make_async_copy(zb1, zh.at[img, pl.ds(hh2, hh2)], s_out1).start()
        return acc

    zero = jnp.zeros((_L,), jnp.float32)
    accs = lax.fori_loop(0, nimg, step, (zero, zero, zero, zero))
    acc = (accs[0] + accs[1]) + (accs[2] + accs[3])

    pltpu.make_async_copy(zb0, zh.at[base + nimg - 1, pl.ds(0, hh2)], s_out0).wait()
    pltpu.make_async_copy(zb1, zh.at[base + nimg - 1, pl.ds(hh2, hh2)], s_out1).wait()
    accv[...] = acc
    pltpu.sync_copy(accv, ph.at[wid])


@functools.partial(jax.jit, static_argnums=(1, 2))
def _pool_sc(xf, h, w):
    nimgs = xf.shape[0]
    h2, w2 = h // 2, w // 2
    nworkers = 32
    nimg = nimgs // nworkers
    mesh = plsc.VectorSubcoreMesh(core_axis_name="c", subcore_axis_name="s")
    body = functools.partial(_sc_body, nimg, h, w)
    return pl.kernel(
        body,
        out_type=(
            jax.ShapeDtypeStruct((nimgs, h2, w2), jnp.float32),
            jax.ShapeDtypeStruct((nworkers, _L), jnp.float32),
        ),
        mesh=mesh,
        compiler_params=pltpu.CompilerParams(
            needs_layout_passes=False, use_tc_tiling_on_sc=True),
        scratch_types=(
            pltpu.VMEM((h // 2, w), jnp.float32),
            pltpu.VMEM((h // 2, w), jnp.float32),
            pltpu.VMEM((h // 4, w2), jnp.float32),
            pltpu.VMEM((h // 4, w2), jnp.float32),
            pltpu.SemaphoreType.DMA,
            pltpu.SemaphoreType.DMA,
            pltpu.SemaphoreType.DMA,
            pltpu.SemaphoreType.DMA,
            pltpu.VMEM((_L,), jnp.float32),
        ),
    )(xf)


def kernel(x):
    b, c, h, w = x.shape
    xf = x.reshape(b * c, h, w)
    zf, partials = _pool_sc(xf, h, w)
    z = zf.reshape(b, c, h // 2, w // 2)
    n = c * (h // 2) * (w // 2)
    const = -0.5 * math.log(2.0 * math.pi) * (3 * n) - math.log(4.0) * n
    ldj = -0.5 * partials.reshape(b, -1).sum(-1) + const
    return (z, ldj)


# speedup vs baseline: 1.3817x; 1.3817x over previous
import functools
import math

import jax
import jax.numpy as jnp
from jax import lax
from jax.experimental import pallas as pl
from jax.experimental.pallas import tpu as pltpu
from jax.experimental.pallas import tpu_sc as plsc

_L = 16


def _sc_body(nimg, h, w, xh, zh, ph, xb0, xb1, zb0, zb1,
             s_in0, s_in1, s_out0, s_out1, accv):
    h2, w2 = h // 2, w // 2
    hh = h // 2
    hh2 = hh // 2
    wid = lax.axis_index("s") * 2 + lax.axis_index("c")
    base = wid * nimg
    lane = lax.iota(jnp.int32, _L)
    evens = [lane * 2 + j * (2 * _L) for j in range(w2 // _L)]

    def compute(xb, zb, accs):
        @plsc.parallel_loop(0, hh2, 1, unroll=1, carry=accs)
        def rp_body(rp, accs):
            aa, ab = accs
            r0 = jnp.full((_L,), 2 * rp, jnp.int32)
            r1 = r0 + 1
            for j in range(w2 // _L):
                odd = evens[j] + 1
                e0 = plsc.load_gather(xb, [r0, evens[j]])
                o0 = plsc.load_gather(xb, [r0, odd])
                e1 = plsc.load_gather(xb, [r1, evens[j]])
                o1 = plsc.load_gather(xb, [r1, odd])
                z = jnp.maximum(jnp.maximum(e0, o0), jnp.maximum(e1, o1))
                zb[rp, pl.ds(j * _L, _L)] = z
                d0 = z - e0
                d1 = z - o0
                d2 = z - e1
                d3 = z - o1
                aa = aa + (d0 * d0 + d1 * d1)
                ab = ab + (d2 * d2 + d3 * d3)
            return (aa, ab)
        return rp_body

    pltpu.make_async_copy(xh.at[base, pl.ds(0, hh)], xb0, s_in0).start()

    def step(g, acc):
        img = base + g
        pltpu.make_async_copy(xh.at[img, pl.ds(hh, hh)], xb1, s_in1).start()
        pltpu.make_async_copy(xh.at[img, pl.ds(0, hh)], xb0, s_in0).wait()

        @pl.when(g > 0)
        def _():
            pltpu.make_async_copy(zb0, zh.at[img - 1, pl.ds(0, hh2)], s_out0).wait()

        acc = compute(xb0, zb0, acc)
        pltpu.make_async_copy(zb0, zh.at[img, pl.ds(0, hh2)], s_out0).start()

        @pl.when(g + 1 < nimg)
        def _():
            pltpu.make_async_copy(xh.at[img + 1, pl.ds(0, hh)], xb0, s_in0).start()

        pltpu.make_async_copy(xh.at[img, pl.ds(hh, hh)], xb1, s_in1).wait()

        @pl.when(g > 0)
        def _():
            pltpu.make_async_copy(zb1, zh.at[img - 1, pl.ds(hh2, hh2)], s_out1).wait()

        acc = compute(xb1, zb1, acc)
        pltpu.make_async_copy(zb1, zh.at[img, pl.ds(hh2, hh2)], s_out1).start()
        return acc

    zero = jnp.zeros((_L,), jnp.float32)
    accs = lax.fori_loop(0, nimg, step, (zero, zero))
    acc = accs[0] + accs[1]

    pltpu.make_async_copy(zb0, zh.at[base + nimg - 1, pl.ds(0, hh2)], s_out0).wait()
    pltpu.make_async_copy(zb1, zh.at[base + nimg - 1, pl.ds(hh2, hh2)], s_out1).wait()
    accv[...] = acc
    pltpu.sync_copy(accv, ph.at[wid])


@functools.partial(jax.jit, static_argnums=(1, 2))
def _pool_sc(xf, h, w):
    nimgs = xf.shape[0]
    h2, w2 = h // 2, w // 2
    nworkers = 32
    nimg = nimgs // nworkers
    mesh = plsc.VectorSubcoreMesh(core_axis_name="c", subcore_axis_name="s")
    body = functools.partial(_sc_body, nimg, h, w)
    return pl.kernel(
        body,
        out_type=(
            jax.ShapeDtypeStruct((nimgs, h2, w2), jnp.float32),
            jax.ShapeDtypeStruct((nworkers, _L), jnp.float32),
        ),
        mesh=mesh,
        compiler_params=pltpu.CompilerParams(
            needs_layout_passes=False, use_tc_tiling_on_sc=True),
        scratch_types=(
            pltpu.VMEM((h // 2, w), jnp.float32),
            pltpu.VMEM((h // 2, w), jnp.float32),
            pltpu.VMEM((h // 4, w2), jnp.float32),
            pltpu.VMEM((h // 4, w2), jnp.float32),
            pltpu.SemaphoreType.DMA,
            pltpu.SemaphoreType.DMA,
            pltpu.SemaphoreType.DMA,
            pltpu.SemaphoreType.DMA,
            pltpu.VMEM((_L,), jnp.float32),
        ),
    )(xf)


def kernel(x):
    b, c, h, w = x.shape
    xf = x.reshape(b * c, h, w)
    zf, partials = _pool_sc(xf, h, w)
    z = zf.reshape(b, c, h // 2, w // 2)
    n = c * (h // 2) * (w // 2)
    const = -0.5 * math.log(2.0 * math.pi) * (3 * n) - math.log(4.0) * n
    ldj = -0.5 * partials.reshape(b, -1).sum(-1) + const
    return (z, ldj)
